# baseline (speedup 1.0000x reference)
import functools

import jax
import jax.numpy as jnp
from jax.experimental import pallas as pl
from jax.experimental.pallas import tpu as pltpu
from jax.experimental.pallas import tpu_sc as plsc

BT = 512
BF = 256
TCH = 512


def _dispatch_vec_kernel(xb_ref, wr_ref, pos_ref, spair_ref, gte_ref, *,
                         n_tiles, bt):
    T = xb_ref.shape[0]
    E = wr_ref.shape[0]
    logits = jax.lax.dot_general(
        xb_ref[...], wr_ref[...], (((1,), (1,)), ((), ())),
        preferred_element_type=jnp.float32)

    ii = jax.lax.broadcasted_iota(jnp.int32, (E, E), 0)
    jj = jax.lax.broadcasted_iota(jnp.int32, (E, E), 1)
    tri_incl = (ii <= jj).astype(jnp.float32)

    def first_occurrence(eq):
        cums = jax.lax.dot_general(eq.astype(jnp.float32), tri_incl,
                                   (((1,), (0,)), ((), ())),
                                   preferred_element_type=jnp.float32)
        return eq & (cums == 1.0)

    m1 = jnp.max(logits, axis=1, keepdims=True)
    fo1 = first_occurrence(logits == m1)
    masked = jnp.where(fo1, -1e30, logits)
    m2 = jnp.max(masked, axis=1, keepdims=True)
    fo2 = first_occurrence(masked == m2)
    o1 = fo1.astype(jnp.float32)
    o2 = fo2.astype(jnp.float32)

    spair_ref[0:T, :] = jax.nn.sigmoid(m1)
    spair_ref[T:2 * T, :] = jax.nn.sigmoid(m2)

    counts = (jnp.sum(o1, axis=0, keepdims=True)
              + jnp.sum(o2, axis=0, keepdims=True))
    pc = jnp.floor((counts + (bt - 1)) / bt) * bt
    tri_strict = (ii < jj).astype(jnp.float32)
    off = jax.lax.dot_general(pc, tri_strict, (((1,), (0,)), ((), ())),
                              preferred_element_type=jnp.float32)

    RB = min(128, T)
    ri = jax.lax.broadcasted_iota(jnp.int32, (RB, RB), 0)
    rj = jax.lax.broadcasted_iota(jnp.int32, (RB, RB), 1)
    ltri = (rj < ri).astype(jnp.float32)
    tot = jnp.zeros((1, E), jnp.float32)
    for b in range(2 * T // RB):
        src = o1 if b < T // RB else o2
        r0 = (b % (T // RB)) * RB
        ob = src[r0:r0 + RB, :]
        cumb = jax.lax.dot_general(ltri, ob, (((1,), (0,)), ((), ())),
                                   preferred_element_type=jnp.float32) + tot
        rank = jnp.sum(ob * cumb, axis=1, keepdims=True)
        offr = jnp.sum(ob * off, axis=1, keepdims=True)
        pos_ref[b * RB:(b + 1) * RB, :] = (rank + offr).astype(jnp.int32)
        tot = tot + jnp.sum(ob, axis=0, keepdims=True)

    ts = (jax.lax.broadcasted_iota(jnp.int32, (n_tiles, E), 0) * bt
          ).astype(jnp.float32)
    ge = (ts >= off).astype(jnp.float32)
    te = jnp.sum(ge, axis=1, keepdims=True) - 1.0
    end = off + counts
    valid = jnp.sum(((ts >= off) & (ts < end)).astype(jnp.float32), axis=1,
                    keepdims=True)
    gte_ref[...] = jnp.where(valid > 0, te, float(E)).astype(jnp.int32)


def _sc_scatter_kernel(pos_hbm, spair_hbm, ids_hbm, sc_hbm, pos_v, sp_v,
                       ids_v, sc_v, *, n_pairs, n_tokens, p_total):
    wid = (jax.lax.axis_index("s") * 2 + jax.lax.axis_index("c"))

    @pl.when(wid == 0)
    def _():
        pltpu.sync_copy(pos_hbm, pos_v)
        pltpu.sync_copy(spair_hbm, sp_v)

        def init(i, _):
            ids_v[pl.ds(i * 16, 16)] = jnp.zeros((16,), jnp.int32)
            sc_v[pl.ds(i * 16, 16)] = jnp.zeros((16,), jnp.float32)
            return _

        jax.lax.fori_loop(0, p_total // 16, init, None, unroll=4)

        def scatter(j, _):
            idx = pos_v[pl.ds(j * 16, 16)]
            tok = jax.lax.broadcasted_iota(jnp.int32, (16,), 0) + j * 16
            tok = jnp.where(tok >= n_tokens, tok - n_tokens, tok)
            plsc.store_scatter(ids_v, [idx], tok)
            plsc.store_scatter(sc_v, [idx], sp_v[pl.ds(j * 16, 16)])
            return _

        jax.lax.fori_loop(0, n_pairs // 16, scatter, None, unroll=4)
        pltpu.sync_copy(ids_v, ids_hbm)
        pltpu.sync_copy(sc_v, sc_hbm)


def _sc_gather_kernel(ids_hbm, x_hbm, xs_hbm, idx_v, rows_v, sem, *, rpw,
                      chunk):
    wid = jax.lax.axis_index("s") * 2 + jax.lax.axis_index("c")
    base = wid * rpw
    pltpu.sync_copy(ids_hbm.at[pl.ds(base, rpw)], idx_v)
    for c in range(rpw // chunk):
        pltpu.async_copy(x_hbm.at[idx_v.at[pl.ds(c * chunk, chunk)]], rows_v,
                         sem).wait()
        pltpu.sync_copy(rows_v, xs_hbm.at[pl.ds(base + c * chunk, chunk)])


def _sc_combine_kernel(pos_hbm, os_hbm, sh_hbm, out_hbm, p0_v, p1_v, r0_v,
                       r1_v, sh_v, o_v, sem, *, tpw, chunk, n_tokens, h):
    wid = jax.lax.axis_index("s") * 2 + jax.lax.axis_index("c")
    tbase = wid * tpw
    pltpu.sync_copy(pos_hbm.at[pl.ds(tbase, tpw)], p0_v)
    pltpu.sync_copy(pos_hbm.at[pl.ds(n_tokens + tbase, tpw)], p1_v)
    for c in range(tpw // chunk):
        pltpu.async_copy(os_hbm.at[p0_v.at[pl.ds(c * chunk, chunk)]], r0_v,
                         sem).wait()
        pltpu.async_copy(os_hbm.at[p1_v.at[pl.ds(c * chunk, chunk)]], r1_v,
                         sem).wait()
        pltpu.sync_copy(sh_hbm.at[pl.ds(tbase + c * chunk, chunk)], sh_v)
        for r in range(chunk):
            def addcol(i, _, r=r):
                sl = pl.ds(i * 16, 16)
                o_v[r, sl] = sh_v[r, sl] + r0_v[r, sl] + r1_v[r, sl]
                return _
            jax.lax.fori_loop(0, h // 16, addcol, None)
        pltpu.sync_copy(o_v, out_hbm.at[pl.ds(tbase + c * chunk, chunk)])


def _routed_lean_kernel(gte_ref, xs_ref, sc_ref, wg_ref, wu_ref, wd_ref,
                        out_ref, acc_ref, *, n_f, n_e):
    g = pl.program_id(0)
    f = pl.program_id(1)
    te = gte_ref[g]
    active = te < n_e

    @pl.when(active)
    def _body():
        wg = wg_ref[0].astype(jnp.bfloat16)
        wu = wu_ref[0].astype(jnp.bfloat16)
        wd = wd_ref[0].astype(jnp.bfloat16)
        xg = xs_ref[...]
        gate = jax.lax.dot_general(xg, wg, (((1,), (1,)), ((), ())),
                                   preferred_element_type=jnp.float32)
        up = jax.lax.dot_general(xg, wu, (((1,), (1,)), ((), ())),
                                 preferred_element_type=jnp.float32)
        act = gate * jax.nn.sigmoid(gate) * up * sc_ref[...]
        actb = act.astype(jnp.bfloat16)
        part = jax.lax.dot_general(actb, wd, (((1,), (1,)), ((), ())),
                                   preferred_element_type=jnp.float32)

        @pl.when(f == 0)
        def _set():
            acc_ref[...] = part

        @pl.when(f > 0)
        def _add():
            acc_ref[...] += part

        @pl.when(f == n_f - 1)
        def _store():
            out_ref[...] = acc_ref[...]


def _routed_kernel(gte_ref, xb_ref, ids_ref, sc_ref, wg_ref, wu_ref, wd_ref,
                   out_ref, xg_ref, acc_ref, *, n_f, n_e, bt, tch):
    g = pl.program_id(0)
    f = pl.program_id(1)
    T = xb_ref.shape[0]
    te = gte_ref[g]
    active = te < n_e

    @pl.when((g == 0) & (f == 0))
    def _init():
        out_ref[...] = jnp.zeros_like(out_ref)

    @pl.when(active)
    def _body():
        @pl.when(f == 0)
        def _gather():
            ids = ids_ref[...]
            for tt in range(T // tch):
                io = jax.lax.broadcasted_iota(jnp.int32, (bt, tch), 1) \
                    + tt * tch
                gm = (io == ids).astype(jnp.bfloat16)
                part = jax.lax.dot_general(
                    gm, xb_ref[tt * tch:(tt + 1) * tch, :],
                    (((1,), (0,)), ((), ())),
                    preferred_element_type=jnp.float32)
                if tt == 0:
                    acc_ref[...] = part
                else:
                    acc_ref[...] += part
            xg_ref[...] = acc_ref[...].astype(jnp.bfloat16)

        wg = wg_ref[0].astype(jnp.bfloat16)
        wu = wu_ref[0].astype(jnp.bfloat16)
        wd = wd_ref[0].astype(jnp.bfloat16)
        xg = xg_ref[...]
        gate = jax.lax.dot_general(xg, wg, (((1,), (1,)), ((), ())),
                                   preferred_element_type=jnp.float32)
        up = jax.lax.dot_general(xg, wu, (((1,), (1,)), ((), ())),
                                 preferred_element_type=jnp.float32)
        act = gate * jax.nn.sigmoid(gate) * up * sc_ref[...]
        actb = act.astype(jnp.bfloat16)
        part = jax.lax.dot_general(actb, wd, (((1,), (1,)), ((), ())),
                                   preferred_element_type=jnp.float32)

        @pl.when(f == 0)
        def _set():
            acc_ref[...] = part

        @pl.when(f > 0)
        def _add():
            acc_ref[...] += part

        @pl.when(f == n_f - 1)
        def _scatter():
            ids = ids_ref[...]
            accb = acc_ref[...].astype(jnp.bfloat16)
            for tt in range(T // tch):
                io = jax.lax.broadcasted_iota(jnp.int32, (bt, tch), 1) \
                    + tt * tch
                gm = (io == ids).astype(jnp.bfloat16)
                out_ref[tt * tch:(tt + 1) * tch, :] += jax.lax.dot_general(
                    gm, accb, (((0,), (0,)), ((), ())),
                    preferred_element_type=jnp.float32)


def _shared_kernel(xb_ref, wgs_ref, wus_ref, wds_ref, out_ref, *, tch):
    f = pl.program_id(0)
    T = xb_ref.shape[0]
    wg = wgs_ref[...].astype(jnp.bfloat16)
    wu = wus_ref[...].astype(jnp.bfloat16)
    wd = wds_ref[...].astype(jnp.bfloat16)
    for tt in range(T // tch):
        sl = slice(tt * tch, (tt + 1) * tch)
        xc = xb_ref[sl, :]
        gate = jax.lax.dot_general(xc, wg, (((1,), (1,)), ((), ())),
                                   preferred_element_type=jnp.float32)
        up = jax.lax.dot_general(xc, wu, (((1,), (1,)), ((), ())),
                                 preferred_element_type=jnp.float32)
        actb = (gate * jax.nn.sigmoid(gate) * up).astype(jnp.bfloat16)
        part = jax.lax.dot_general(actb, wd, (((1,), (1,)), ((), ())),
                                   preferred_element_type=jnp.float32)

        @pl.when(f == 0)
        def _set():
            out_ref[sl, :] = part

        @pl.when(f > 0)
        def _add():
            out_ref[sl, :] += part


def kernel(hidden_states, W_router, Wg_experts, Wu_experts, Wd_experts,
           Wg_shared, Wu_shared, Wd_shared):
    T, H = hidden_states.shape
    E, F, _ = Wg_experts.shape
    bt = min(BT, T)
    bf = min(BF, F)
    tch = min(TCH, T)
    n_f = F // bf
    n_pairs = 2 * T
    p_total = 2 * T + E * bt
    n_tiles = p_total // bt

    xb = hidden_states.astype(jnp.bfloat16)
    wrb = W_router.astype(jnp.bfloat16)

    pos, spair, gte = pl.pallas_call(
        functools.partial(_dispatch_vec_kernel, n_tiles=n_tiles, bt=bt),
        grid=(1,),
        in_specs=[
            pl.BlockSpec((T, H), lambda i: (0, 0)),
            pl.BlockSpec((E, H), lambda i: (0, 0)),
        ],
        out_specs=[
            pl.BlockSpec((n_pairs, 1), lambda i: (0, 0)),
            pl.BlockSpec((n_pairs, 1), lambda i: (0, 0)),
            pl.BlockSpec((n_tiles, 1), lambda i: (0, 0)),
        ],
        out_shape=[
            jax.ShapeDtypeStruct((n_pairs, 1), jnp.int32),
            jax.ShapeDtypeStruct((n_pairs, 1), jnp.float32),
            jax.ShapeDtypeStruct((n_tiles, 1), jnp.int32),
        ],
        compiler_params=pltpu.CompilerParams(
            dimension_semantics=("arbitrary",)),
    )(xb, wrb)

    sc_mesh = plsc.VectorSubcoreMesh(core_axis_name="c", subcore_axis_name="s")
    ids_col, sc_col = pl.kernel(
        functools.partial(_sc_scatter_kernel, n_pairs=n_pairs, n_tokens=T,
                          p_total=p_total),
        out_type=[
            jax.ShapeDtypeStruct((p_total,), jnp.int32),
            jax.ShapeDtypeStruct((p_total,), jnp.float32),
        ],
        mesh=sc_mesh,
        compiler_params=pltpu.CompilerParams(needs_layout_passes=False),
        scratch_types=[
            pltpu.VMEM((n_pairs,), jnp.int32),
            pltpu.VMEM((n_pairs,), jnp.float32),
            pltpu.VMEM((p_total,), jnp.int32),
            pltpu.VMEM((p_total,), jnp.float32),
        ],
    )(pos.reshape(n_pairs), spair.reshape(n_pairs))
    ids_col = ids_col.reshape(p_total, 1)
    sc_col = sc_col.reshape(p_total, 1)

    x_i32 = jax.lax.bitcast_convert_type(
        xb.reshape(T, H // 2, 2), jnp.int32)
    rpw = p_total // 32
    gchunk = min(64, rpw)
    xs_i32 = pl.kernel(
        functools.partial(_sc_gather_kernel, rpw=rpw, chunk=gchunk),
        out_type=jax.ShapeDtypeStruct((p_total, H // 2), jnp.int32),
        mesh=plsc.VectorSubcoreMesh(core_axis_name="c",
                                    subcore_axis_name="s"),
        compiler_params=pltpu.CompilerParams(needs_layout_passes=False),
        scratch_types=[
            pltpu.VMEM((rpw,), jnp.int32),
            pltpu.VMEM((gchunk, H // 2), jnp.int32),
            pltpu.SemaphoreType.DMA,
        ],
    )(ids_col.reshape(p_total), x_i32)
    xs = jax.lax.bitcast_convert_type(xs_i32, jnp.bfloat16).reshape(
        p_total, H)

    bfl = min(512, F)
    n_fl = F // bfl
    grid_spec = pltpu.PrefetchScalarGridSpec(
        num_scalar_prefetch=1,
        grid=(n_tiles, n_fl),
        in_specs=[
            pl.BlockSpec((bt, H), lambda g, f, m: (g, 0)),
            pl.BlockSpec((bt, 1), lambda g, f, m: (g, 0)),
            pl.BlockSpec((1, bfl, H), lambda g, f, m: (
                jnp.minimum(m[g], E - 1),
                jnp.where(m[g] < E, f, 0), 0)),
            pl.BlockSpec((1, bfl, H), lambda g, f, m: (
                jnp.minimum(m[g], E - 1),
                jnp.where(m[g] < E, f, 0), 0)),
            pl.BlockSpec((1, H, bfl), lambda g, f, m: (
                jnp.minimum(m[g], E - 1), 0,
                jnp.where(m[g] < E, f, 0))),
        ],
        out_specs=pl.BlockSpec((bt, H), lambda g, f, m: (g, 0)),
        scratch_shapes=[pltpu.VMEM((bt, H), jnp.float32)],
    )
    out_sorted = pl.pallas_call(
        functools.partial(_routed_lean_kernel, n_f=n_fl, n_e=E),
        grid_spec=grid_spec,
        out_shape=jax.ShapeDtypeStruct((p_total, H), jnp.float32),
        compiler_params=pltpu.CompilerParams(
            dimension_semantics=("arbitrary", "arbitrary")),
    )(gte.reshape(n_tiles), xs, sc_col, Wg_experts, Wu_experts, Wd_experts)

    shared_out = pl.pallas_call(
        functools.partial(_shared_kernel, tch=tch),
        grid=(n_f,),
        in_specs=[
            pl.BlockSpec((T, H), lambda f: (0, 0)),
            pl.BlockSpec((bf, H), lambda f: (f, 0)),
            pl.BlockSpec((bf, H), lambda f: (f, 0)),
            pl.BlockSpec((H, bf), lambda f: (0, f)),
        ],
        out_specs=pl.BlockSpec((T, H), lambda f: (0, 0)),
        out_shape=jax.ShapeDtypeStruct((T, H), jnp.float32),
        compiler_params=pltpu.CompilerParams(
            dimension_semantics=("arbitrary",)),
    )(xb, Wg_shared, Wu_shared, Wd_shared)

    tpw = T // 32
    cchunk = min(8, tpw)
    final = pl.kernel(
        functools.partial(_sc_combine_kernel, tpw=tpw, chunk=cchunk,
                          n_tokens=T, h=H),
        out_type=jax.ShapeDtypeStruct((T, H), jnp.float32),
        mesh=plsc.VectorSubcoreMesh(core_axis_name="c",
                                    subcore_axis_name="s"),
        compiler_params=pltpu.CompilerParams(needs_layout_passes=False),
        scratch_types=[
            pltpu.VMEM((tpw,), jnp.int32),
            pltpu.VMEM((tpw,), jnp.int32),
            pltpu.VMEM((cchunk, H), jnp.float32),
            pltpu.VMEM((cchunk, H), jnp.float32),
            pltpu.VMEM((cchunk, H), jnp.float32),
            pltpu.VMEM((cchunk, H), jnp.float32),
            pltpu.SemaphoreType.DMA,
        ],
    )(pos.reshape(n_pairs), out_sorted, shared_out)
    return final


# speedup vs baseline: 1.7651x; 1.7651x over previous
import functools

import jax
import jax.numpy as jnp
from jax.experimental import pallas as pl
from jax.experimental.pallas import tpu as pltpu
from jax.experimental.pallas import tpu_sc as plsc

BT = 512
BF = 256
TCH = 512


def _dispatch_vec_kernel(xb_ref, wr_ref, pos_ref, spair_ref, gte_ref, *,
                         n_tiles, bt):
    T = xb_ref.shape[0]
    E = wr_ref.shape[0]
    logits = jax.lax.dot_general(
        xb_ref[...], wr_ref[...], (((1,), (1,)), ((), ())),
        preferred_element_type=jnp.float32)

    ii = jax.lax.broadcasted_iota(jnp.int32, (E, E), 0)
    jj = jax.lax.broadcasted_iota(jnp.int32, (E, E), 1)
    tri_incl = (ii <= jj).astype(jnp.float32)

    def first_occurrence(eq):
        cums = jax.lax.dot_general(eq.astype(jnp.float32), tri_incl,
                                   (((1,), (0,)), ((), ())),
                                   preferred_element_type=jnp.float32)
        return eq & (cums == 1.0)

    m1 = jnp.max(logits, axis=1, keepdims=True)
    fo1 = first_occurrence(logits == m1)
    masked = jnp.where(fo1, -1e30, logits)
    m2 = jnp.max(masked, axis=1, keepdims=True)
    fo2 = first_occurrence(masked == m2)
    o1 = fo1.astype(jnp.float32)
    o2 = fo2.astype(jnp.float32)

    spair_ref[0:T, :] = jax.nn.sigmoid(m1)
    spair_ref[T:2 * T, :] = jax.nn.sigmoid(m2)

    counts = (jnp.sum(o1, axis=0, keepdims=True)
              + jnp.sum(o2, axis=0, keepdims=True))
    pc = jnp.floor((counts + (bt - 1)) / bt) * bt
    tri_strict = (ii < jj).astype(jnp.float32)
    off = jax.lax.dot_general(pc, tri_strict, (((1,), (0,)), ((), ())),
                              preferred_element_type=jnp.float32)

    RB = min(128, T)
    ri = jax.lax.broadcasted_iota(jnp.int32, (RB, RB), 0)
    rj = jax.lax.broadcasted_iota(jnp.int32, (RB, RB), 1)
    ltri = (rj < ri).astype(jnp.float32)
    nb = 2 * T // RB
    bsums = []
    for b in range(nb):
        src = o1 if b < T // RB else o2
        r0 = (b % (T // RB)) * RB
        ob = src[r0:r0 + RB, :]
        cumb = jax.lax.dot_general(ltri, ob, (((1,), (0,)), ((), ())),
                                   preferred_element_type=jnp.float32)
        rank = jnp.sum(ob * cumb, axis=1, keepdims=True)
        offr = jnp.sum(ob * off, axis=1, keepdims=True)
        pos_ref[b * RB:(b + 1) * RB, :] = (rank + offr).astype(jnp.int32)
        bsums.append(jnp.sum(ob, axis=0, keepdims=True))
    bs = jnp.concatenate(bsums, axis=0)
    bi = jax.lax.broadcasted_iota(jnp.int32, (nb, nb), 0)
    bj = jax.lax.broadcasted_iota(jnp.int32, (nb, nb), 1)
    btri = (bj < bi).astype(jnp.float32)
    btp = jax.lax.dot_general(btri, bs, (((1,), (0,)), ((), ())),
                              preferred_element_type=jnp.float32)
    for b in range(nb):
        src = o1 if b < T // RB else o2
        r0 = (b % (T // RB)) * RB
        ob = src[r0:r0 + RB, :]
        fix = jnp.sum(ob * btp[b:b + 1, :], axis=1, keepdims=True)
        pos_ref[b * RB:(b + 1) * RB, :] += fix.astype(jnp.int32)

    ts = (jax.lax.broadcasted_iota(jnp.int32, (n_tiles, E), 0) * bt
          ).astype(jnp.float32)
    ge = (ts >= off).astype(jnp.float32)
    te = jnp.sum(ge, axis=1, keepdims=True) - 1.0
    end = off + counts
    valid = jnp.sum(((ts >= off) & (ts < end)).astype(jnp.float32), axis=1,
                    keepdims=True)
    gte_ref[...] = jnp.where(valid > 0, te, float(E)).astype(jnp.int32)


def _sc_scatter_kernel(pos_hbm, spair_hbm, ids_hbm, sc_hbm, pos_v, sp_v,
                       ids_v, sc_v, *, n_pairs, n_tokens, p_total):
    wid = (jax.lax.axis_index("s") * 2 + jax.lax.axis_index("c"))

    @pl.when(wid == 0)
    def _():
        pltpu.sync_copy(pos_hbm, pos_v)
        pltpu.sync_copy(spair_hbm, sp_v)

        def init(i, _):
            ids_v[pl.ds(i * 16, 16)] = jnp.zeros((16,), jnp.int32)
            sc_v[pl.ds(i * 16, 16)] = jnp.zeros((16,), jnp.float32)
            return _

        jax.lax.fori_loop(0, p_total // 16, init, None, unroll=4)

        def scatter(j, _):
            idx = pos_v[pl.ds(j * 16, 16)]
            tok = jax.lax.broadcasted_iota(jnp.int32, (16,), 0) + j * 16
            tok = jnp.where(tok >= n_tokens, tok - n_tokens, tok)
            plsc.store_scatter(ids_v, [idx], tok)
            plsc.store_scatter(sc_v, [idx], sp_v[pl.ds(j * 16, 16)])
            return _

        jax.lax.fori_loop(0, n_pairs // 16, scatter, None, unroll=4)
        pltpu.sync_copy(ids_v, ids_hbm)
        pltpu.sync_copy(sc_v, sc_hbm)


def _routed_kernel(gte_ref, xb_ref, ids_ref, sc_ref, wg_ref, wu_ref, wd_ref,
                   out_ref, xg_ref, acc_ref, *, n_f, n_e, bt, tch):
    g = pl.program_id(0)
    f = pl.program_id(1)
    T = xb_ref.shape[0]
    te = gte_ref[g]
    active = te < n_e

    @pl.when((g == 0) & (f == 0))
    def _init():
        out_ref[...] = jnp.zeros_like(out_ref)

    @pl.when(active)
    def _body():
        @pl.when(f == 0)
        def _gather():
            ids = ids_ref[...]
            for tt in range(T // tch):
                io = jax.lax.broadcasted_iota(jnp.int32, (bt, tch), 1) \
                    + tt * tch
                gm = (io == ids).astype(jnp.bfloat16)
                part = jax.lax.dot_general(
                    gm, xb_ref[tt * tch:(tt + 1) * tch, :],
                    (((1,), (0,)), ((), ())),
                    preferred_element_type=jnp.float32)
                if tt == 0:
                    acc_ref[...] = part
                else:
                    acc_ref[...] += part
            xg_ref[...] = acc_ref[...].astype(jnp.bfloat16)

        wg = wg_ref[0].astype(jnp.bfloat16)
        wu = wu_ref[0].astype(jnp.bfloat16)
        wd = wd_ref[0].astype(jnp.bfloat16)
        xg = xg_ref[...]
        gate = jax.lax.dot_general(xg, wg, (((1,), (1,)), ((), ())),
                                   preferred_element_type=jnp.float32)
        up = jax.lax.dot_general(xg, wu, (((1,), (1,)), ((), ())),
                                 preferred_element_type=jnp.float32)
        act = gate * jax.nn.sigmoid(gate) * up * sc_ref[...]
        actb = act.astype(jnp.bfloat16)
        part = jax.lax.dot_general(actb, wd, (((1,), (1,)), ((), ())),
                                   preferred_element_type=jnp.float32)

        @pl.when(f == 0)
        def _set():
            acc_ref[...] = part

        @pl.when(f > 0)
        def _add():
            acc_ref[...] += part

        @pl.when(f == n_f - 1)
        def _scatter():
            ids = ids_ref[...]
            accb = acc_ref[...].astype(jnp.bfloat16)
            for tt in range(T // tch):
                io = jax.lax.broadcasted_iota(jnp.int32, (bt, tch), 1) \
                    + tt * tch
                gm = (io == ids).astype(jnp.bfloat16)
                out_ref[tt * tch:(tt + 1) * tch, :] += jax.lax.dot_general(
                    gm, accb, (((0,), (0,)), ((), ())),
                    preferred_element_type=jnp.float32)


def _shared_kernel(xb_ref, wgs_ref, wus_ref, wds_ref, out_ref, *, tch):
    f = pl.program_id(0)
    T = xb_ref.shape[0]
    wg = wgs_ref[...].astype(jnp.bfloat16)
    wu = wus_ref[...].astype(jnp.bfloat16)
    wd = wds_ref[...].astype(jnp.bfloat16)
    for tt in range(T // tch):
        sl = slice(tt * tch, (tt + 1) * tch)
        xc = xb_ref[sl, :]
        gate = jax.lax.dot_general(xc, wg, (((1,), (1,)), ((), ())),
                                   preferred_element_type=jnp.float32)
        up = jax.lax.dot_general(xc, wu, (((1,), (1,)), ((), ())),
                                 preferred_element_type=jnp.float32)
        actb = (gate * jax.nn.sigmoid(gate) * up).astype(jnp.bfloat16)
        part = jax.lax.dot_general(actb, wd, (((1,), (1,)), ((), ())),
                                   preferred_element_type=jnp.float32)

        @pl.when(f == 0)
        def _set():
            out_ref[sl, :] = part

        @pl.when(f > 0)
        def _add():
            out_ref[sl, :] += part


def kernel(hidden_states, W_router, Wg_experts, Wu_experts, Wd_experts,
           Wg_shared, Wu_shared, Wd_shared):
    T, H = hidden_states.shape
    E, F, _ = Wg_experts.shape
    bt = min(BT, T)
    bf = min(BF, F)
    tch = min(TCH, T)
    n_f = F // bf
    n_pairs = 2 * T
    p_total = 2 * T + E * bt
    n_tiles = p_total // bt

    xb = hidden_states.astype(jnp.bfloat16)
    wrb = W_router.astype(jnp.bfloat16)

    pos, spair, gte = pl.pallas_call(
        functools.partial(_dispatch_vec_kernel, n_tiles=n_tiles, bt=bt),
        grid=(1,),
        in_specs=[
            pl.BlockSpec((T, H), lambda i: (0, 0)),
            pl.BlockSpec((E, H), lambda i: (0, 0)),
        ],
        out_specs=[
            pl.BlockSpec((n_pairs, 1), lambda i: (0, 0)),
            pl.BlockSpec((n_pairs, 1), lambda i: (0, 0)),
            pl.BlockSpec((n_tiles, 1), lambda i: (0, 0)),
        ],
        out_shape=[
            jax.ShapeDtypeStruct((n_pairs, 1), jnp.int32),
            jax.ShapeDtypeStruct((n_pairs, 1), jnp.float32),
            jax.ShapeDtypeStruct((n_tiles, 1), jnp.int32),
        ],
        compiler_params=pltpu.CompilerParams(
            dimension_semantics=("arbitrary",)),
    )(xb, wrb)

    sc_mesh = plsc.VectorSubcoreMesh(core_axis_name="c", subcore_axis_name="s")
    ids_col, sc_col = pl.kernel(
        functools.partial(_sc_scatter_kernel, n_pairs=n_pairs, n_tokens=T,
                          p_total=p_total),
        out_type=[
            jax.ShapeDtypeStruct((p_total,), jnp.int32),
            jax.ShapeDtypeStruct((p_total,), jnp.float32),
        ],
        mesh=sc_mesh,
        compiler_params=pltpu.CompilerParams(needs_layout_passes=False),
        scratch_types=[
            pltpu.VMEM((n_pairs,), jnp.int32),
            pltpu.VMEM((n_pairs,), jnp.float32),
            pltpu.VMEM((p_total,), jnp.int32),
            pltpu.VMEM((p_total,), jnp.float32),
        ],
    )(pos.reshape(n_pairs), spair.reshape(n_pairs))
    ids_col = ids_col.reshape(p_total, 1)
    sc_col = sc_col.reshape(p_total, 1)

    grid_spec = pltpu.PrefetchScalarGridSpec(
        num_scalar_prefetch=1,
        grid=(n_tiles, n_f),
        in_specs=[
            pl.BlockSpec((T, H), lambda g, f, m: (0, 0)),
            pl.BlockSpec((bt, 1), lambda g, f, m: (g, 0)),
            pl.BlockSpec((bt, 1), lambda g, f, m: (g, 0)),
            pl.BlockSpec((1, bf, H), lambda g, f, m: (
                jnp.minimum(m[g], E - 1),
                jnp.where(m[g] < E, f, 0), 0)),
            pl.BlockSpec((1, bf, H), lambda g, f, m: (
                jnp.minimum(m[g], E - 1),
                jnp.where(m[g] < E, f, 0), 0)),
            pl.BlockSpec((1, H, bf), lambda g, f, m: (
                jnp.minimum(m[g], E - 1), 0,
                jnp.where(m[g] < E, f, 0))),
        ],
        out_specs=pl.BlockSpec((T, H), lambda g, f, m: (0, 0)),
        scratch_shapes=[
            pltpu.VMEM((bt, H), jnp.bfloat16),
            pltpu.VMEM((bt, H), jnp.float32),
        ],
    )
    routed_out = pl.pallas_call(
        functools.partial(_routed_kernel, n_f=n_f, n_e=E, bt=bt, tch=tch),
        grid_spec=grid_spec,
        out_shape=jax.ShapeDtypeStruct((T, H), jnp.float32),
        compiler_params=pltpu.CompilerParams(
            dimension_semantics=("arbitrary", "arbitrary")),
    )(gte.reshape(n_tiles), xb, ids_col, sc_col, Wg_experts,
      Wu_experts, Wd_experts)

    shared_out = pl.pallas_call(
        functools.partial(_shared_kernel, tch=tch),
        grid=(n_f,),
        in_specs=[
            pl.BlockSpec((T, H), lambda f: (0, 0)),
            pl.BlockSpec((bf, H), lambda f: (f, 0)),
            pl.BlockSpec((bf, H), lambda f: (f, 0)),
            pl.BlockSpec((H, bf), lambda f: (0, f)),
        ],
        out_specs=pl.BlockSpec((T, H), lambda f: (0, 0)),
        out_shape=jax.ShapeDtypeStruct((T, H), jnp.float32),
        compiler_params=pltpu.CompilerParams(
            dimension_semantics=("arbitrary",)),
    )(xb, Wg_shared, Wu_shared, Wd_shared)

    _PROBE_B = False
    if _PROBE_B:
        return shared_out + gte[:1, :1].astype(jnp.float32) * 1e-30
    return routed_out + shared_out


# speedup vs baseline: 1.9442x; 1.1015x over previous
import functools

import jax
import jax.numpy as jnp
from jax.experimental import pallas as pl
from jax.experimental.pallas import tpu as pltpu
from jax.experimental.pallas import tpu_sc as plsc

BT = 512
BF = 512
TCH = 512


def _dispatch_vec_kernel(xb_ref, wr_ref, pos_ref, spair_ref, gte_ref, *,
                         n_tiles, bt):
    T = xb_ref.shape[0]
    E = wr_ref.shape[0]
    logits = jax.lax.dot_general(
        xb_ref[...], wr_ref[...], (((1,), (1,)), ((), ())),
        preferred_element_type=jnp.float32)

    ii = jax.lax.broadcasted_iota(jnp.int32, (E, E), 0)
    jj = jax.lax.broadcasted_iota(jnp.int32, (E, E), 1)
    tri_incl = (ii <= jj).astype(jnp.float32)

    def first_occurrence(eq):
        cums = jax.lax.dot_general(eq.astype(jnp.float32), tri_incl,
                                   (((1,), (0,)), ((), ())),
                                   preferred_element_type=jnp.float32)
        return eq & (cums == 1.0)

    m1 = jnp.max(logits, axis=1, keepdims=True)
    fo1 = first_occurrence(logits == m1)
    masked = jnp.where(fo1, -1e30, logits)
    m2 = jnp.max(masked, axis=1, keepdims=True)
    fo2 = first_occurrence(masked == m2)
    o1 = fo1.astype(jnp.float32)
    o2 = fo2.astype(jnp.float32)

    spair_ref[0:T, :] = jax.nn.sigmoid(m1)
    spair_ref[T:2 * T, :] = jax.nn.sigmoid(m2)

    counts = (jnp.sum(o1, axis=0, keepdims=True)
              + jnp.sum(o2, axis=0, keepdims=True))
    pc = jnp.floor((counts + (bt - 1)) / bt) * bt
    tri_strict = (ii < jj).astype(jnp.float32)
    off = jax.lax.dot_general(pc, tri_strict, (((1,), (0,)), ((), ())),
                              preferred_element_type=jnp.float32)

    RB = min(128, T)
    ri = jax.lax.broadcasted_iota(jnp.int32, (RB, RB), 0)
    rj = jax.lax.broadcasted_iota(jnp.int32, (RB, RB), 1)
    ltri = (rj < ri).astype(jnp.float32)
    nb = 2 * T // RB
    bsums = []
    for b in range(nb):
        src = o1 if b < T // RB else o2
        r0 = (b % (T // RB)) * RB
        ob = src[r0:r0 + RB, :]
        cumb = jax.lax.dot_general(ltri, ob, (((1,), (0,)), ((), ())),
                                   preferred_element_type=jnp.float32)
        rank = jnp.sum(ob * cumb, axis=1, keepdims=True)
        offr = jnp.sum(ob * off, axis=1, keepdims=True)
        pos_ref[b * RB:(b + 1) * RB, :] = (rank + offr).astype(jnp.int32)
        bsums.append(jnp.sum(ob, axis=0, keepdims=True))
    bs = jnp.concatenate(bsums, axis=0)
    bi = jax.lax.broadcasted_iota(jnp.int32, (nb, nb), 0)
    bj = jax.lax.broadcasted_iota(jnp.int32, (nb, nb), 1)
    btri = (bj < bi).astype(jnp.float32)
    btp = jax.lax.dot_general(btri, bs, (((1,), (0,)), ((), ())),
                              preferred_element_type=jnp.float32)
    for b in range(nb):
        src = o1 if b < T // RB else o2
        r0 = (b % (T // RB)) * RB
        ob = src[r0:r0 + RB, :]
        fix = jnp.sum(ob * btp[b:b + 1, :], axis=1, keepdims=True)
        pos_ref[b * RB:(b + 1) * RB, :] += fix.astype(jnp.int32)

    ts = (jax.lax.broadcasted_iota(jnp.int32, (n_tiles, E), 0) * bt
          ).astype(jnp.float32)
    ge = (ts >= off).astype(jnp.float32)
    te = jnp.sum(ge, axis=1, keepdims=True) - 1.0
    end = off + counts
    valid = jnp.sum(((ts >= off) & (ts < end)).astype(jnp.float32), axis=1,
                    keepdims=True)
    gte_ref[...] = jnp.where(valid > 0, te, float(E)).astype(jnp.int32)


def _sc_scatter_kernel(pos_hbm, spair_hbm, ids_hbm, sc_hbm, pos_v, sp_v,
                       ids_v, sc_v, *, n_pairs, n_tokens, p_total):
    wid = (jax.lax.axis_index("s") * 2 + jax.lax.axis_index("c"))

    @pl.when(wid == 0)
    def _():
        pltpu.sync_copy(pos_hbm, pos_v)
        pltpu.sync_copy(spair_hbm, sp_v)

        def init(i, _):
            ids_v[pl.ds(i * 16, 16)] = jnp.zeros((16,), jnp.int32)
            sc_v[pl.ds(i * 16, 16)] = jnp.zeros((16,), jnp.float32)
            return _

        jax.lax.fori_loop(0, p_total // 16, init, None, unroll=4)

        def scatter(j, _):
            idx = pos_v[pl.ds(j * 16, 16)]
            tok = jax.lax.broadcasted_iota(jnp.int32, (16,), 0) + j * 16
            tok = jnp.where(tok >= n_tokens, tok - n_tokens, tok)
            plsc.store_scatter(ids_v, [idx], tok)
            plsc.store_scatter(sc_v, [idx], sp_v[pl.ds(j * 16, 16)])
            return _

        jax.lax.fori_loop(0, n_pairs // 16, scatter, None, unroll=4)
        pltpu.sync_copy(ids_v, ids_hbm)
        pltpu.sync_copy(sc_v, sc_hbm)


def _routed_kernel(gte_ref, xb_ref, ids_ref, sc_ref, wg_ref, wu_ref, wd_ref,
                   out_ref, xg_ref, acc_ref, *, n_f, n_e, bt, tch):
    g = pl.program_id(0)
    f = pl.program_id(1)
    T = xb_ref.shape[0]
    te = gte_ref[g]
    active = te < n_e

    @pl.when((g == 0) & (f == 0))
    def _init():
        out_ref[...] = jnp.zeros_like(out_ref)

    @pl.when(active)
    def _body():
        @pl.when(f == 0)
        def _gather():
            ids = ids_ref[...]
            for tt in range(T // tch):
                io = jax.lax.broadcasted_iota(jnp.int32, (bt, tch), 1) \
                    + tt * tch
                gm = (io == ids).astype(jnp.bfloat16)
                part = jax.lax.dot_general(
                    gm, xb_ref[tt * tch:(tt + 1) * tch, :],
                    (((1,), (0,)), ((), ())),
                    preferred_element_type=jnp.float32)
                if tt == 0:
                    acc_ref[...] = part
                else:
                    acc_ref[...] += part
            xg_ref[...] = acc_ref[...].astype(jnp.bfloat16)

        wg = wg_ref[0].astype(jnp.bfloat16)
        wu = wu_ref[0].astype(jnp.bfloat16)
        wd = wd_ref[0].astype(jnp.bfloat16)
        xg = xg_ref[...]
        gate = jax.lax.dot_general(xg, wg, (((1,), (1,)), ((), ())),
                                   preferred_element_type=jnp.float32)
        up = jax.lax.dot_general(xg, wu, (((1,), (1,)), ((), ())),
                                 preferred_element_type=jnp.float32)
        act = gate * jax.nn.sigmoid(gate) * up * sc_ref[...]
        actb = act.astype(jnp.bfloat16)
        part = jax.lax.dot_general(actb, wd, (((1,), (1,)), ((), ())),
                                   preferred_element_type=jnp.float32)

        @pl.when(f == 0)
        def _set():
            acc_ref[...] = part

        @pl.when(f > 0)
        def _add():
            acc_ref[...] += part

        @pl.when(f == n_f - 1)
        def _scatter():
            ids = ids_ref[...]
            accb = acc_ref[...].astype(jnp.bfloat16)
            for tt in range(T // tch):
                io = jax.lax.broadcasted_iota(jnp.int32, (bt, tch), 1) \
                    + tt * tch
                gm = (io == ids).astype(jnp.bfloat16)
                out_ref[tt * tch:(tt + 1) * tch, :] += jax.lax.dot_general(
                    gm, accb, (((0,), (0,)), ((), ())),
                    preferred_element_type=jnp.float32)


def _shared_kernel(xb_ref, wgs_ref, wus_ref, wds_ref, out_ref, *, tch):
    f = pl.program_id(0)
    T = xb_ref.shape[0]
    wg = wgs_ref[...].astype(jnp.bfloat16)
    wu = wus_ref[...].astype(jnp.bfloat16)
    wd = wds_ref[...].astype(jnp.bfloat16)
    for tt in range(T // tch):
        sl = slice(tt * tch, (tt + 1) * tch)
        xc = xb_ref[sl, :]
        gate = jax.lax.dot_general(xc, wg, (((1,), (1,)), ((), ())),
                                   preferred_element_type=jnp.float32)
        up = jax.lax.dot_general(xc, wu, (((1,), (1,)), ((), ())),
                                 preferred_element_type=jnp.float32)
        actb = (gate * jax.nn.sigmoid(gate) * up).astype(jnp.bfloat16)
        part = jax.lax.dot_general(actb, wd, (((1,), (1,)), ((), ())),
                                   preferred_element_type=jnp.float32)

        @pl.when(f == 0)
        def _set():
            out_ref[sl, :] = part

        @pl.when(f > 0)
        def _add():
            out_ref[sl, :] += part


def kernel(hidden_states, W_router, Wg_experts, Wu_experts, Wd_experts,
           Wg_shared, Wu_shared, Wd_shared):
    T, H = hidden_states.shape
    E, F, _ = Wg_experts.shape
    bt = min(BT, T)
    bf = min(BF, F)
    tch = min(TCH, T)
    n_f = F // bf
    n_pairs = 2 * T
    p_total = 2 * T + E * bt
    n_tiles = p_total // bt

    xb = hidden_states.astype(jnp.bfloat16)
    wrb = W_router.astype(jnp.bfloat16)

    pos, spair, gte = pl.pallas_call(
        functools.partial(_dispatch_vec_kernel, n_tiles=n_tiles, bt=bt),
        grid=(1,),
        in_specs=[
            pl.BlockSpec((T, H), lambda i: (0, 0)),
            pl.BlockSpec((E, H), lambda i: (0, 0)),
        ],
        out_specs=[
            pl.BlockSpec((n_pairs, 1), lambda i: (0, 0)),
            pl.BlockSpec((n_pairs, 1), lambda i: (0, 0)),
            pl.BlockSpec((n_tiles, 1), lambda i: (0, 0)),
        ],
        out_shape=[
            jax.ShapeDtypeStruct((n_pairs, 1), jnp.int32),
            jax.ShapeDtypeStruct((n_pairs, 1), jnp.float32),
            jax.ShapeDtypeStruct((n_tiles, 1), jnp.int32),
        ],
        compiler_params=pltpu.CompilerParams(
            dimension_semantics=("arbitrary",)),
    )(xb, wrb)

    sc_mesh = plsc.VectorSubcoreMesh(core_axis_name="c", subcore_axis_name="s")
    ids_col, sc_col = pl.kernel(
        functools.partial(_sc_scatter_kernel, n_pairs=n_pairs, n_tokens=T,
                          p_total=p_total),
        out_type=[
            jax.ShapeDtypeStruct((p_total,), jnp.int32),
            jax.ShapeDtypeStruct((p_total,), jnp.float32),
        ],
        mesh=sc_mesh,
        compiler_params=pltpu.CompilerParams(needs_layout_passes=False),
        scratch_types=[
            pltpu.VMEM((n_pairs,), jnp.int32),
            pltpu.VMEM((n_pairs,), jnp.float32),
            pltpu.VMEM((p_total,), jnp.int32),
            pltpu.VMEM((p_total,), jnp.float32),
        ],
    )(pos.reshape(n_pairs), spair.reshape(n_pairs))
    ids_col = ids_col.reshape(p_total, 1)
    sc_col = sc_col.reshape(p_total, 1)

    grid_spec = pltpu.PrefetchScalarGridSpec(
        num_scalar_prefetch=1,
        grid=(n_tiles, n_f),
        in_specs=[
            pl.BlockSpec((T, H), lambda g, f, m: (0, 0)),
            pl.BlockSpec((bt, 1), lambda g, f, m: (g, 0)),
            pl.BlockSpec((bt, 1), lambda g, f, m: (g, 0)),
            pl.BlockSpec((1, bf, H), lambda g, f, m: (
                jnp.minimum(m[g], E - 1),
                jnp.where(m[g] < E, f, 0), 0)),
            pl.BlockSpec((1, bf, H), lambda g, f, m: (
                jnp.minimum(m[g], E - 1),
                jnp.where(m[g] < E, f, 0), 0)),
            pl.BlockSpec((1, H, bf), lambda g, f, m: (
                jnp.minimum(m[g], E - 1), 0,
                jnp.where(m[g] < E, f, 0))),
        ],
        out_specs=pl.BlockSpec((T, H), lambda g, f, m: (0, 0)),
        scratch_shapes=[
            pltpu.VMEM((bt, H), jnp.bfloat16),
            pltpu.VMEM((bt, H), jnp.float32),
        ],
    )
    routed_out = pl.pallas_call(
        functools.partial(_routed_kernel, n_f=n_f, n_e=E, bt=bt, tch=tch),
        grid_spec=grid_spec,
        out_shape=jax.ShapeDtypeStruct((T, H), jnp.float32),
        compiler_params=pltpu.CompilerParams(
            dimension_semantics=("arbitrary", "arbitrary"),
            vmem_limit_bytes=64 * 1024 * 1024),
    )(gte.reshape(n_tiles), xb, ids_col, sc_col, Wg_experts,
      Wu_experts, Wd_experts)

    shared_out = pl.pallas_call(
        functools.partial(_shared_kernel, tch=tch),
        grid=(n_f,),
        in_specs=[
            pl.BlockSpec((T, H), lambda f: (0, 0)),
            pl.BlockSpec((bf, H), lambda f: (f, 0)),
            pl.BlockSpec((bf, H), lambda f: (f, 0)),
            pl.BlockSpec((H, bf), lambda f: (0, f)),
        ],
        out_specs=pl.BlockSpec((T, H), lambda f: (0, 0)),
        out_shape=jax.ShapeDtypeStruct((T, H), jnp.float32),
        compiler_params=pltpu.CompilerParams(
            dimension_semantics=("arbitrary",)),
    )(xb, Wg_shared, Wu_shared, Wd_shared)

    _PROBE_B = False
    if _PROBE_B:
        return shared_out + gte[:1, :1].astype(jnp.float32) * 1e-30
    return routed_out + shared_out
